# baseline (speedup 1.0000x reference)
import functools

import jax
import jax.numpy as jnp
from jax import lax
from jax.experimental import pallas as pl
from jax.experimental.pallas import tpu as pltpu
from jax.experimental.pallas import tpu_sc as plsc

D_MODEL = 1024
SEQ_LEN = 4096
BATCH = 4
NUM_CORES = 2
NUM_SUBCORES = 16
NUM_WORKERS = NUM_CORES * NUM_SUBCORES
SEQ_PER_WORKER = SEQ_LEN // NUM_WORKERS
CHUNK_ROWS = 16
CHUNK_ELEMS = CHUNK_ROWS * D_MODEL
NUM_CHUNKS = SEQ_PER_WORKER // CHUNK_ROWS
LANES = 16
VECS_PER_CHUNK = CHUNK_ELEMS // LANES


def _sc_body(x_hbm, emb_hbm, out_hbm, emb_v, x_v):
    wid = lax.axis_index("s") * NUM_CORES + lax.axis_index("c")
    base = wid * (SEQ_PER_WORKER * D_MODEL)

    def chunk_body(k, _):
        off = base + k * CHUNK_ELEMS
        pltpu.sync_copy(emb_hbm.at[pl.ds(off, CHUNK_ELEMS)], emb_v)
        for b in range(BATCH):
            pltpu.sync_copy(x_hbm.at[b, pl.ds(off, CHUNK_ELEMS)], x_v)

            def add_body(v, _):
                sl = pl.ds(v * LANES, LANES)
                x_v[sl] = x_v[sl] + emb_v[sl]
                return 0

            lax.fori_loop(0, VECS_PER_CHUNK, add_body, 0)
            pltpu.sync_copy(x_v, out_hbm.at[b, pl.ds(off, CHUNK_ELEMS)])
        return 0

    lax.fori_loop(0, NUM_CHUNKS, chunk_body, 0)


@jax.jit
def _pos_emb_add(x2, emb1):
    mesh = plsc.VectorSubcoreMesh(core_axis_name="c", subcore_axis_name="s")
    fn = pl.kernel(
        _sc_body,
        mesh=mesh,
        out_type=jax.ShapeDtypeStruct((BATCH, SEQ_LEN * D_MODEL), jnp.float32),
        scratch_types=[
            pltpu.VMEM((CHUNK_ELEMS,), jnp.float32),
            pltpu.VMEM((CHUNK_ELEMS,), jnp.float32),
        ],
    )
    return fn(x2, emb1)


def kernel(x, emb_table):
    seq_len = x.shape[1]
    x2 = x.reshape(BATCH, seq_len * D_MODEL)
    emb1 = emb_table[:seq_len].reshape(seq_len * D_MODEL)
    out = _pos_emb_add(x2, emb1)
    return out.reshape(x.shape)


# speedup vs baseline: 1.8850x; 1.8850x over previous
import functools

import jax
import jax.numpy as jnp
from jax import lax
from jax.experimental import pallas as pl
from jax.experimental.pallas import tpu as pltpu
from jax.experimental.pallas import tpu_sc as plsc

D_MODEL = 1024
SEQ_LEN = 4096
BATCH = 4
NUM_CORES = 2
NUM_SUBCORES = 16
NUM_WORKERS = NUM_CORES * NUM_SUBCORES
SEQ_PER_WORKER = SEQ_LEN // NUM_WORKERS
CHUNK_ROWS = 16
CHUNK_ELEMS = CHUNK_ROWS * D_MODEL
NUM_CHUNKS = SEQ_PER_WORKER // CHUNK_ROWS
LANES = 16
VECS_PER_CHUNK = CHUNK_ELEMS // LANES
UNROLL = 8
NSTEPS = NUM_CHUNKS * BATCH


def _sc_body(x_hbm, emb_hbm, out_hbm,
             xb0, xb1, xb2, xb3, eb0, eb1,
             ls0, ls1, ls2, ls3, ss0, ss1, ss2, ss3, es0, es1):
    xbufs = [xb0, xb1, xb2, xb3]
    ebufs = [eb0, eb1]
    lsems = [ls0, ls1, ls2, ls3]
    ssems = [ss0, ss1, ss2, ss3]
    esems = [es0, es1]

    wid = lax.axis_index("s") * NUM_CORES + lax.axis_index("c")
    base = wid * (SEQ_PER_WORKER * D_MODEL)

    def x_off(step):
        k, b = divmod(step, BATCH)
        return b, base + k * CHUNK_ELEMS

    def issue_load(step):
        b, off = x_off(step)
        j = step % 4
        return pltpu.async_copy(
            x_hbm.at[b, pl.ds(off, CHUNK_ELEMS)], xbufs[j], lsems[j])

    def issue_emb(k):
        return pltpu.async_copy(
            emb_hbm.at[pl.ds(base + k * CHUNK_ELEMS, CHUNK_ELEMS)],
            ebufs[k % 2], esems[k % 2])

    def issue_store(step):
        b, off = x_off(step)
        j = step % 4
        return pltpu.async_copy(
            xbufs[j], out_hbm.at[b, pl.ds(off, CHUNK_ELEMS)], ssems[j])

    pending_load = [None] * 4
    pending_store = [None] * 4
    pending_emb = [None] * 2

    pending_emb[0] = issue_emb(0)
    pending_load[0] = issue_load(0)
    pending_load[1] = issue_load(1)

    for s in range(NSTEPS):
        j = s % 4
        k = s // BATCH
        if s % BATCH == 0 and k + 1 < NUM_CHUNKS:
            pending_emb[(k + 1) % 2] = issue_emb(k + 1)
        if s + 2 < NSTEPS:
            if pending_store[(s + 2) % 4] is not None:
                pending_store[(s + 2) % 4].wait()
                pending_store[(s + 2) % 4] = None
            pending_load[(s + 2) % 4] = issue_load(s + 2)
        if s % BATCH == 0:
            pending_emb[k % 2].wait()
            pending_emb[k % 2] = None
        pending_load[j].wait()
        pending_load[j] = None

        xv = xbufs[j]
        ev = ebufs[k % 2]

        def add_body(i, _, xv=xv, ev=ev):
            v0 = i * (UNROLL * LANES)
            for u in range(UNROLL):
                sl = pl.ds(v0 + u * LANES, LANES)
                xv[sl] = xv[sl] + ev[sl]
            return 0

        lax.fori_loop(0, VECS_PER_CHUNK // UNROLL, add_body, 0)
        pending_store[j] = issue_store(s)

    for j in range(4):
        if pending_store[j] is not None:
            pending_store[j].wait()


@jax.jit
def _pos_emb_add(x2, emb1):
    mesh = plsc.VectorSubcoreMesh(core_axis_name="c", subcore_axis_name="s")
    fn = pl.kernel(
        _sc_body,
        mesh=mesh,
        out_type=jax.ShapeDtypeStruct((BATCH, SEQ_LEN * D_MODEL), jnp.float32),
        scratch_types=[pltpu.VMEM((CHUNK_ELEMS,), jnp.float32)] * 4
        + [pltpu.VMEM((CHUNK_ELEMS,), jnp.float32)] * 2
        + [pltpu.SemaphoreType.DMA] * 10,
    )
    return fn(x2, emb1)


def kernel(x, emb_table):
    seq_len = x.shape[1]
    x2 = x.reshape(BATCH, seq_len * D_MODEL)
    emb1 = emb_table[:seq_len].reshape(seq_len * D_MODEL)
    out = _pos_emb_add(x2, emb1)
    return out.reshape(x.shape)
